# baseline (speedup 1.0000x reference)
import functools

import jax
import jax.numpy as jnp
from jax import lax
from jax.experimental import pallas as pl
from jax.experimental.pallas import tpu as pltpu
from jax.experimental.pallas import tpu_sc as plsc

E = 8
K = 2
D = 1024
F = 512
SF = 2048
T = 2048
P = T * K

TILE = 512
_PADMAX = (E * (TILE - 1)) // TILE * TILE
NSLOT = P + _PADMAX
NTILES = NSLOT // TILE

NC = 2
NS = 16
NW = NC * NS
RPW = P // NW
CH = 64
NCH = RPW // CH
CHS = 32
NCHS = RPW // CHS


def _cumsum_lanes(a):
    n = a.shape[1]
    sh = 1
    while sh < n:
        a = a + jnp.concatenate(
            [jnp.zeros((a.shape[0], sh), a.dtype), a[:, :n - sh]], axis=1)
        sh *= 2
    return a


def _cumsum_rows(a):
    n = a.shape[0]
    sh = 1
    while sh < n:
        a = a + jnp.concatenate(
            [jnp.zeros((sh, a.shape[1]), a.dtype), a[:n - sh, :]], axis=0)
        sh *= 2
    return a


def _router_plan_body(x_ref, rw_ref, wpair_ref, slot_ref, texp_ref,
                      par_ref, nxe_ref):
    x = x_ref[...]
    rw = rw_ref[...]
    lg = lax.dot_general(rw, x, (((1,), (1,)), ((), ())),
                         preferred_element_type=jnp.float32)
    ie = lax.broadcasted_iota(jnp.int32, (E, T), 0)
    m1 = jnp.max(lg, axis=0, keepdims=True)
    i1 = jnp.min(jnp.where(lg == m1, ie, E), axis=0, keepdims=True)
    lg2 = jnp.where(ie == i1, -1e30, lg)
    m2 = jnp.max(lg2, axis=0, keepdims=True)
    i2 = jnp.min(jnp.where(lg2 == m2, ie, E), axis=0, keepdims=True)
    b = jnp.exp(m2 - m1)
    w0 = 1.0 / (1.0 + b)
    wpair_ref[...] = jnp.concatenate([w0, b * w0], axis=1)

    oh = jnp.concatenate([(ie == i1), (ie == i2)], axis=1).astype(jnp.float32)
    csum = _cumsum_lanes(oh)
    rank = jnp.sum((csum - oh) * oh, axis=0, keepdims=True)
    counts = csum[:, P - 1:P].astype(jnp.int32)
    padded = ((counts + (TILE - 1)) // TILE) * TILE
    offinc = _cumsum_rows(padded)
    off = offinc - padded
    offpick = jnp.sum(oh * off.astype(jnp.float32), axis=0, keepdims=True)
    slot_ref[...] = (rank + offpick).astype(jnp.int32)

    ntile_used = offinc[E - 1:E, :] // TILE
    jt = lax.broadcasted_iota(jnp.int32, (1, NTILES), 1)
    texp = jnp.zeros((1, NTILES), jnp.int32)
    boff = off // TILE
    for e in range(1, E):
        texp = texp + (jt >= boff[e:e + 1, :]).astype(jnp.int32)
    texp = jnp.where(jt < ntile_used, texp, E)
    texp_ref[...] = texp

    chg = jnp.concatenate(
        [jnp.ones((1, 1), jnp.float32),
         (texp[:, 1:] != texp[:, :-1]).astype(jnp.float32)], axis=1)
    runidx = _cumsum_lanes(chg).astype(jnp.int32) - 1
    par_ref[...] = jnp.bitwise_and(runidx, 1)
    eidx = lax.broadcasted_iota(jnp.int32, (E, NTILES), 0)
    present = (padded > 0)
    cand = jnp.where((eidx > texp) & present, eidx, E)
    nxe_ref[...] = jnp.min(cand, axis=0, keepdims=True)


def _router_plan(x, rw):
    return pl.pallas_call(
        _router_plan_body,
        out_shape=[
            jax.ShapeDtypeStruct((1, P), jnp.float32),
            jax.ShapeDtypeStruct((1, P), jnp.int32),
            jax.ShapeDtypeStruct((1, NTILES), jnp.int32),
            jax.ShapeDtypeStruct((1, NTILES), jnp.int32),
            jax.ShapeDtypeStruct((1, NTILES), jnp.int32),
        ],
    )(x, rw)


def _sc_dispatch(x, slots):
    mesh = plsc.VectorSubcoreMesh(core_axis_name="c", subcore_axis_name="s")

    @functools.partial(
        pl.kernel,
        out_type=jax.ShapeDtypeStruct((NSLOT, D), jnp.float32),
        mesh=mesh,
        scratch_types=[
            pltpu.VMEM((CHS, D), jnp.float32),
            pltpu.VMEM((CHS, D), jnp.float32),
            pltpu.VMEM((CHS,), jnp.int32),
            pltpu.VMEM((CHS,), jnp.int32),
            pltpu.SemaphoreType.DMA((2,)),
            pltpu.SemaphoreType.DMA((2,)),
        ],
    )
    def k(x_hbm, slots_hbm, xs_hbm, row_a, row_b, idx_a, idx_b, rsem, ssem):
        wid = lax.axis_index("s") * NC + lax.axis_index("c")
        base = wid * RPW
        basex = pl.multiple_of(base - jnp.where(base >= T, T, 0), RPW)
        rows = (row_a, row_b)
        idxs = (idx_a, idx_b)

        pend = [None, None]
        for c in range(NCHS):
            b = c % 2
            if pend[b] is not None:
                pend[b].wait()
            pltpu.sync_copy(slots_hbm.at[pl.ds(base + c * CHS, CHS)], idxs[b])
            r = pltpu.async_copy(
                x_hbm.at[pl.ds(basex + c * CHS, CHS)], rows[b], rsem.at[b])
            r.wait()
            pend[b] = pltpu.async_copy(rows[b], xs_hbm.at[idxs[b]],
                                       ssem.at[b])
        pend[0].wait()
        pend[1].wait()

    return k(x, slots)


def _ffn_body(te_ref, par_ref, nxe_ref, xs_ref, wg_hbm, wu_hbm, wd_hbm,
              ds_ref, wg_s, wu_s, wd_s, sems):
    j = pl.program_id(0)
    te = te_ref[j]
    par = par_ref[j]
    nxe = nxe_ref[j]
    prev = te_ref[jnp.maximum(j, 1) - 1]
    chg = (j == 0) | (te != prev)

    def copies(e, b):
        return (pltpu.make_async_copy(wg_hbm.at[e], wg_s.at[b], sems.at[b, 0]),
                pltpu.make_async_copy(wu_hbm.at[e], wu_s.at[b], sems.at[b, 1]),
                pltpu.make_async_copy(wd_hbm.at[e], wd_s.at[b], sems.at[b, 2]))

    @pl.when(chg & (te < E))
    def _():
        @pl.when(j == 0)
        def _():
            for cp in copies(te, par):
                cp.start()

        for cp in copies(te, par):
            cp.wait()

        @pl.when(nxe < E)
        def _():
            for cp in copies(jnp.minimum(nxe, E - 1), 1 - par):
                cp.start()

    @pl.when(te < E)
    def _():
        xt = xs_ref[...].astype(jnp.bfloat16)
        wg = wg_s[par].astype(jnp.bfloat16)
        wu = wu_s[par].astype(jnp.bfloat16)
        wd = wd_s[par].astype(jnp.bfloat16)
        g = lax.dot_general(xt, wg, (((1,), (1,)), ((), ())),
                            preferred_element_type=jnp.float32)
        u = lax.dot_general(xt, wu, (((1,), (1,)), ((), ())),
                            preferred_element_type=jnp.float32)
        h = ((g * jax.nn.sigmoid(g)) * u).astype(jnp.bfloat16)
        dn = lax.dot_general(h, wd, (((1,), (1,)), ((), ())),
                             preferred_element_type=jnp.float32)
        ub = lax.bitcast_convert_type(dn, jnp.uint32)
        rb = (ub + jnp.uint32(0x7FFF) + ((ub >> 16) & jnp.uint32(1))) >> 16
        ds_ref[...] = rb[:, :D // 2] | (rb[:, D // 2:] << 16)


def _ffn_grid_spec():
    return pltpu.PrefetchScalarGridSpec(
        num_scalar_prefetch=3,
        grid=(NTILES,),
        in_specs=[
            pl.BlockSpec((TILE, D), lambda j, te, pr, nx: (j, 0)),
            pl.BlockSpec(memory_space=pl.ANY),
            pl.BlockSpec(memory_space=pl.ANY),
            pl.BlockSpec(memory_space=pl.ANY),
        ],
        out_specs=pl.BlockSpec((TILE, D // 2), lambda j, te, pr, nx: (j, 0)),
        scratch_shapes=[
            pltpu.VMEM((2, F, D), jnp.float32),
            pltpu.VMEM((2, F, D), jnp.float32),
            pltpu.VMEM((2, D, F), jnp.float32),
            pltpu.SemaphoreType.DMA((2, 3)),
        ],
    )


def _ffn(texp, par, nxe, xs, w_gate, w_up, w_down):
    return pl.pallas_call(
        _ffn_body,
        grid_spec=_ffn_grid_spec(),
        out_shape=jax.ShapeDtypeStruct((NSLOT, D // 2), jnp.uint32),
    )(texp, par, nxe, xs, w_gate, w_up, w_down)


def _sc_combine(ds, slots):
    mesh = plsc.VectorSubcoreMesh(core_axis_name="c", subcore_axis_name="s")

    @functools.partial(
        pl.kernel,
        out_type=[
            jax.ShapeDtypeStruct((T, D // 2), jnp.uint32),
            jax.ShapeDtypeStruct((T, D // 2), jnp.uint32),
        ],
        mesh=mesh,
        scratch_types=[
            pltpu.VMEM((CH,), jnp.int32),
            pltpu.VMEM((CH, D // 2), jnp.uint32),
            pltpu.SemaphoreType.DMA,
        ],
    )
    def k(ds_hbm, slots_hbm, g0_hbm, g1_hbm, idx_v, row_v, sem):
        wid = lax.axis_index("s") * NC + lax.axis_index("c")
        base = wid * RPW
        for c in range(NCH):
            lo = base + c * CH
            pltpu.sync_copy(slots_hbm.at[pl.ds(lo, CH)], idx_v)
            pltpu.async_copy(ds_hbm.at[idx_v], row_v, sem).wait()

            @pl.when(base < T)
            def _():
                pltpu.sync_copy(row_v, g0_hbm.at[pl.ds(lo, CH)])

            @pl.when(base >= T)
            def _():
                pltpu.sync_copy(
                    row_v,
                    g1_hbm.at[pl.ds(pl.multiple_of(jnp.maximum(lo - T, 0), CH),
                                    CH)])

    return k(ds, slots)


_SH_TILE = 256


def _shared_body(x_ref, swg_ref, swu_ref, swd_ref, sgw_ref, o_ref):
    xt = x_ref[...]
    xb = xt.astype(jnp.bfloat16)
    g = lax.dot_general(xb, swg_ref[...].astype(jnp.bfloat16),
                        (((1,), (1,)), ((), ())),
                        preferred_element_type=jnp.float32)
    u = lax.dot_general(xb, swu_ref[...].astype(jnp.bfloat16),
                        (((1,), (1,)), ((), ())),
                        preferred_element_type=jnp.float32)
    sh = ((g * jax.nn.sigmoid(g)) * u).astype(jnp.bfloat16)
    dn = lax.dot_general(sh, swd_ref[...].astype(jnp.bfloat16),
                         (((1,), (1,)), ((), ())),
                         preferred_element_type=jnp.float32)
    gate = jax.nn.sigmoid(
        lax.dot_general(xt, sgw_ref[...], (((1,), (1,)), ((), ())),
                        preferred_element_type=jnp.float32))
    o_ref[...] = gate * dn


def _shared(x, sw_gate, sw_up, sw_down, shared_gate_w):
    return pl.pallas_call(
        _shared_body,
        grid=(T // _SH_TILE,),
        in_specs=[
            pl.BlockSpec((_SH_TILE, D), lambda j: (j, 0)),
            pl.BlockSpec((SF, D), lambda j: (0, 0)),
            pl.BlockSpec((SF, D), lambda j: (0, 0)),
            pl.BlockSpec((D, SF), lambda j: (0, 0)),
            pl.BlockSpec((1, D), lambda j: (0, 0)),
        ],
        out_specs=pl.BlockSpec((_SH_TILE, D), lambda j: (j, 0)),
        out_shape=jax.ShapeDtypeStruct((T, D), jnp.float32),
    )(x, sw_gate, sw_up, sw_down, shared_gate_w)


_CB_TILE = 512


def _unpack_g(ref):
    p = ref[...]
    lo = lax.bitcast_convert_type(p << 16, jnp.float32)
    hi = lax.bitcast_convert_type(p & jnp.uint32(0xFFFF0000), jnp.float32)
    return jnp.concatenate([lo, hi], axis=1)


def _combine_body(g0_ref, g1_ref, w0_ref, w1_ref, shg_ref, o_ref):
    o_ref[...] = (w0_ref[...] * _unpack_g(g0_ref)
                  + w1_ref[...] * _unpack_g(g1_ref) + shg_ref[...])


def _combine(g0, g1, w0, w1, shg):
    return pl.pallas_call(
        _combine_body,
        grid=(T // _CB_TILE,),
        in_specs=[
            pl.BlockSpec((_CB_TILE, D // 2), lambda j: (j, 0)),
            pl.BlockSpec((_CB_TILE, D // 2), lambda j: (j, 0)),
            pl.BlockSpec((_CB_TILE, 1), lambda j: (j, 0)),
            pl.BlockSpec((_CB_TILE, 1), lambda j: (j, 0)),
            pl.BlockSpec((_CB_TILE, D), lambda j: (j, 0)),
        ],
        out_specs=pl.BlockSpec((_CB_TILE, D), lambda j: (j, 0)),
        out_shape=jax.ShapeDtypeStruct((T, D), jnp.float32),
    )(g0, g1, w0, w1, shg)


def kernel(hidden_states, router_weight, w_gate, w_up, w_down,
           sw_gate, sw_up, sw_down, shared_gate_w):
    b, s, d = hidden_states.shape
    x = hidden_states.reshape(-1, d)

    wpair, slot2d, texp2d, par2d, nxe2d = _router_plan(x, router_weight)
    slots = slot2d.reshape(P)
    texp = texp2d.reshape(NTILES)
    par = par2d.reshape(NTILES)
    nxe = nxe2d.reshape(NTILES)
    wp = wpair.reshape(P, 1)

    xs = _sc_dispatch(x, slots)
    ds = _ffn(texp, par, nxe, xs, w_gate, w_up, w_down)
    g0, g1 = _sc_combine(ds, slots)

    shg = _shared(x, sw_gate, sw_up, sw_down, shared_gate_w)
    out = _combine(g0, g1, wp[:T], wp[T:], shg)
    return out.reshape(b, s, d)


# speedup vs baseline: 1.0342x; 1.0342x over previous
import functools

import jax
import jax.numpy as jnp
from jax import lax
from jax.experimental import pallas as pl
from jax.experimental.pallas import tpu as pltpu
from jax.experimental.pallas import tpu_sc as plsc

E = 8
K = 2
D = 1024
F = 512
SF = 2048
T = 2048
P = T * K

TILE = 256
_PADMAX = (E * (TILE - 1)) // TILE * TILE
NSLOT = P + _PADMAX
NTILES = NSLOT // TILE

NC = 2
NS = 16
NW = NC * NS
RPW = P // NW
CH = 64
NCH = RPW // CH


def _cumsum_lanes(a):
    n = a.shape[1]
    sh = 1
    while sh < n:
        a = a + jnp.concatenate(
            [jnp.zeros((a.shape[0], sh), a.dtype), a[:, :n - sh]], axis=1)
        sh *= 2
    return a


def _cumsum_rows(a):
    n = a.shape[0]
    sh = 1
    while sh < n:
        a = a + jnp.concatenate(
            [jnp.zeros((sh, a.shape[1]), a.dtype), a[:n - sh, :]], axis=0)
        sh *= 2
    return a


def _router_plan_body(x_ref, rw_ref, wpair_ref, slot_ref, texp_ref,
                      par_ref, nxe_ref):
    x = x_ref[...]
    rw = rw_ref[...]
    lg = lax.dot_general(rw, x, (((1,), (1,)), ((), ())),
                         preferred_element_type=jnp.float32)
    ie = lax.broadcasted_iota(jnp.int32, (E, T), 0)
    m1 = jnp.max(lg, axis=0, keepdims=True)
    i1 = jnp.min(jnp.where(lg == m1, ie, E), axis=0, keepdims=True)
    lg2 = jnp.where(ie == i1, -1e30, lg)
    m2 = jnp.max(lg2, axis=0, keepdims=True)
    i2 = jnp.min(jnp.where(lg2 == m2, ie, E), axis=0, keepdims=True)
    b = jnp.exp(m2 - m1)
    w0 = 1.0 / (1.0 + b)
    wpair_ref[...] = jnp.concatenate([w0, b * w0], axis=1)

    oh = jnp.concatenate([(ie == i1), (ie == i2)], axis=1).astype(jnp.float32)
    csum = _cumsum_lanes(oh)
    rank = jnp.sum((csum - oh) * oh, axis=0, keepdims=True)
    counts = csum[:, P - 1:P].astype(jnp.int32)
    padded = ((counts + (TILE - 1)) // TILE) * TILE
    offinc = _cumsum_rows(padded)
    off = offinc - padded
    offpick = jnp.sum(oh * off.astype(jnp.float32), axis=0, keepdims=True)
    slot_ref[...] = (rank + offpick).astype(jnp.int32)

    ntile_used = offinc[E - 1:E, :] // TILE
    jt = lax.broadcasted_iota(jnp.int32, (1, NTILES), 1)
    texp = jnp.zeros((1, NTILES), jnp.int32)
    boff = off // TILE
    for e in range(1, E):
        texp = texp + (jt >= boff[e:e + 1, :]).astype(jnp.int32)
    texp = jnp.where(jt < ntile_used, texp, E)
    texp_ref[...] = texp

    chg = jnp.concatenate(
        [jnp.ones((1, 1), jnp.float32),
         (texp[:, 1:] != texp[:, :-1]).astype(jnp.float32)], axis=1)
    runidx = _cumsum_lanes(chg).astype(jnp.int32) - 1
    par_ref[...] = jnp.bitwise_and(runidx, 1)
    eidx = lax.broadcasted_iota(jnp.int32, (E, NTILES), 0)
    present = (padded > 0)
    cand = jnp.where((eidx > texp) & present, eidx, E)
    nxe_ref[...] = jnp.min(cand, axis=0, keepdims=True)


def _router_plan(x, rw):
    return pl.pallas_call(
        _router_plan_body,
        out_shape=[
            jax.ShapeDtypeStruct((1, P), jnp.float32),
            jax.ShapeDtypeStruct((1, P), jnp.int32),
            jax.ShapeDtypeStruct((1, NTILES), jnp.int32),
            jax.ShapeDtypeStruct((1, NTILES), jnp.int32),
            jax.ShapeDtypeStruct((1, NTILES), jnp.int32),
        ],
    )(x, rw)


def _sc_dispatch(x, slots):
    mesh = plsc.VectorSubcoreMesh(core_axis_name="c", subcore_axis_name="s")

    @functools.partial(
        pl.kernel,
        out_type=jax.ShapeDtypeStruct((NSLOT, D), jnp.float32),
        mesh=mesh,
        scratch_types=[
            pltpu.VMEM((CH,), jnp.int32),
            pltpu.VMEM((CH, D), jnp.float32),
            pltpu.SemaphoreType.DMA,
        ],
    )
    def k(x_hbm, slots_hbm, xs_hbm, idx_v, row_v, sem):
        wid = lax.axis_index("s") * NC + lax.axis_index("c")
        base = wid * RPW
        basex = pl.multiple_of(base - jnp.where(base >= T, T, 0), RPW)
        for c in range(NCH):
            lo = base + c * CH
            pltpu.sync_copy(slots_hbm.at[pl.ds(lo, CH)], idx_v)
            pltpu.sync_copy(x_hbm.at[pl.ds(basex + c * CH, CH)], row_v)
            pltpu.async_copy(row_v, xs_hbm.at[idx_v], sem).wait()

    return k(x, slots)


def _ffn_body(te_ref, par_ref, nxe_ref, xs_ref, wg_hbm, wu_hbm, wd_hbm,
              ds_ref, wg_s, wu_s, wd_s, sems):
    j = pl.program_id(0)
    te = te_ref[j]
    par = par_ref[j]
    nxe = nxe_ref[j]
    prev = te_ref[jnp.maximum(j, 1) - 1]
    chg = (j == 0) | (te != prev)

    def copies(e, b):
        return (pltpu.make_async_copy(wg_hbm.at[e], wg_s.at[b], sems.at[b, 0]),
                pltpu.make_async_copy(wu_hbm.at[e], wu_s.at[b], sems.at[b, 1]),
                pltpu.make_async_copy(wd_hbm.at[e], wd_s.at[b], sems.at[b, 2]))

    @pl.when(chg & (te < E))
    def _():
        @pl.when(j == 0)
        def _():
            for cp in copies(te, par):
                cp.start()

        for cp in copies(te, par):
            cp.wait()

        @pl.when(nxe < E)
        def _():
            for cp in copies(jnp.minimum(nxe, E - 1), 1 - par):
                cp.start()

    @pl.when(te < E)
    def _():
        xt = xs_ref[...].astype(jnp.bfloat16)
        wg = wg_s[par].astype(jnp.bfloat16)
        wu = wu_s[par].astype(jnp.bfloat16)
        wd = wd_s[par].astype(jnp.bfloat16)
        g = lax.dot_general(xt, wg, (((1,), (1,)), ((), ())),
                            preferred_element_type=jnp.float32)
        u = lax.dot_general(xt, wu, (((1,), (1,)), ((), ())),
                            preferred_element_type=jnp.float32)
        h = ((g * jax.nn.sigmoid(g)) * u).astype(jnp.bfloat16)
        dn = lax.dot_general(h, wd, (((1,), (1,)), ((), ())),
                             preferred_element_type=jnp.float32)
        ub = lax.bitcast_convert_type(dn, jnp.uint32)
        rb = (ub + jnp.uint32(0x7FFF) + ((ub >> 16) & jnp.uint32(1))) >> 16
        ds_ref[...] = rb[:, :D // 2] | (rb[:, D // 2:] << 16)


def _ffn_grid_spec():
    return pltpu.PrefetchScalarGridSpec(
        num_scalar_prefetch=3,
        grid=(NTILES,),
        in_specs=[
            pl.BlockSpec((TILE, D), lambda j, te, pr, nx: (j, 0)),
            pl.BlockSpec(memory_space=pl.ANY),
            pl.BlockSpec(memory_space=pl.ANY),
            pl.BlockSpec(memory_space=pl.ANY),
        ],
        out_specs=pl.BlockSpec((TILE, D // 2), lambda j, te, pr, nx: (j, 0)),
        scratch_shapes=[
            pltpu.VMEM((2, F, D), jnp.float32),
            pltpu.VMEM((2, F, D), jnp.float32),
            pltpu.VMEM((2, D, F), jnp.float32),
            pltpu.SemaphoreType.DMA((2, 3)),
        ],
    )


def _ffn(texp, par, nxe, xs, w_gate, w_up, w_down):
    return pl.pallas_call(
        _ffn_body,
        grid_spec=_ffn_grid_spec(),
        out_shape=jax.ShapeDtypeStruct((NSLOT, D // 2), jnp.uint32),
    )(texp, par, nxe, xs, w_gate, w_up, w_down)


def _sc_combine(ds, slots):
    mesh = plsc.VectorSubcoreMesh(core_axis_name="c", subcore_axis_name="s")

    @functools.partial(
        pl.kernel,
        out_type=[
            jax.ShapeDtypeStruct((T, D // 2), jnp.uint32),
            jax.ShapeDtypeStruct((T, D // 2), jnp.uint32),
        ],
        mesh=mesh,
        scratch_types=[
            pltpu.VMEM((CH,), jnp.int32),
            pltpu.VMEM((CH, D // 2), jnp.uint32),
            pltpu.SemaphoreType.DMA,
        ],
    )
    def k(ds_hbm, slots_hbm, g0_hbm, g1_hbm, idx_v, row_v, sem):
        wid = lax.axis_index("s") * NC + lax.axis_index("c")
        base = wid * RPW
        for c in range(NCH):
            lo = base + c * CH
            pltpu.sync_copy(slots_hbm.at[pl.ds(lo, CH)], idx_v)
            pltpu.async_copy(ds_hbm.at[idx_v], row_v, sem).wait()

            @pl.when(base < T)
            def _():
                pltpu.sync_copy(row_v, g0_hbm.at[pl.ds(lo, CH)])

            @pl.when(base >= T)
            def _():
                pltpu.sync_copy(
                    row_v,
                    g1_hbm.at[pl.ds(pl.multiple_of(jnp.maximum(lo - T, 0), CH),
                                    CH)])

    return k(ds, slots)


_SH_TILE = 256


def _shared_body(x_ref, swg_ref, swu_ref, swd_ref, sgw_ref, o_ref):
    xt = x_ref[...]
    xb = xt.astype(jnp.bfloat16)
    g = lax.dot_general(xb, swg_ref[...].astype(jnp.bfloat16),
                        (((1,), (1,)), ((), ())),
                        preferred_element_type=jnp.float32)
    u = lax.dot_general(xb, swu_ref[...].astype(jnp.bfloat16),
                        (((1,), (1,)), ((), ())),
                        preferred_element_type=jnp.float32)
    sh = ((g * jax.nn.sigmoid(g)) * u).astype(jnp.bfloat16)
    dn = lax.dot_general(sh, swd_ref[...].astype(jnp.bfloat16),
                         (((1,), (1,)), ((), ())),
                         preferred_element_type=jnp.float32)
    gate = jax.nn.sigmoid(
        lax.dot_general(xt, sgw_ref[...], (((1,), (1,)), ((), ())),
                        preferred_element_type=jnp.float32))
    o_ref[...] = gate * dn


def _shared(x, sw_gate, sw_up, sw_down, shared_gate_w):
    return pl.pallas_call(
        _shared_body,
        grid=(T // _SH_TILE,),
        in_specs=[
            pl.BlockSpec((_SH_TILE, D), lambda j: (j, 0)),
            pl.BlockSpec((SF, D), lambda j: (0, 0)),
            pl.BlockSpec((SF, D), lambda j: (0, 0)),
            pl.BlockSpec((D, SF), lambda j: (0, 0)),
            pl.BlockSpec((1, D), lambda j: (0, 0)),
        ],
        out_specs=pl.BlockSpec((_SH_TILE, D), lambda j: (j, 0)),
        out_shape=jax.ShapeDtypeStruct((T, D), jnp.float32),
    )(x, sw_gate, sw_up, sw_down, shared_gate_w)


_CB_TILE = 512


def _unpack_g(ref):
    p = ref[...]
    lo = lax.bitcast_convert_type(p << 16, jnp.float32)
    hi = lax.bitcast_convert_type(p & jnp.uint32(0xFFFF0000), jnp.float32)
    return jnp.concatenate([lo, hi], axis=1)


def _combine_body(g0_ref, g1_ref, w0_ref, w1_ref, shg_ref, o_ref):
    o_ref[...] = (w0_ref[...] * _unpack_g(g0_ref)
                  + w1_ref[...] * _unpack_g(g1_ref) + shg_ref[...])


def _combine(g0, g1, w0, w1, shg):
    return pl.pallas_call(
        _combine_body,
        grid=(T // _CB_TILE,),
        in_specs=[
            pl.BlockSpec((_CB_TILE, D // 2), lambda j: (j, 0)),
            pl.BlockSpec((_CB_TILE, D // 2), lambda j: (j, 0)),
            pl.BlockSpec((_CB_TILE, 1), lambda j: (j, 0)),
            pl.BlockSpec((_CB_TILE, 1), lambda j: (j, 0)),
            pl.BlockSpec((_CB_TILE, D), lambda j: (j, 0)),
        ],
        out_specs=pl.BlockSpec((_CB_TILE, D), lambda j: (j, 0)),
        out_shape=jax.ShapeDtypeStruct((T, D), jnp.float32),
    )(g0, g1, w0, w1, shg)


def kernel(hidden_states, router_weight, w_gate, w_up, w_down,
           sw_gate, sw_up, sw_down, shared_gate_w):
    b, s, d = hidden_states.shape
    x = hidden_states.reshape(-1, d)

    wpair, slot2d, texp2d, par2d, nxe2d = _router_plan(x, router_weight)
    slots = slot2d.reshape(P)
    texp = texp2d.reshape(NTILES)
    par = par2d.reshape(NTILES)
    nxe = nxe2d.reshape(NTILES)
    wp = wpair.reshape(P, 1)

    xs = _sc_dispatch(x, slots)
    ds = _ffn(texp, par, nxe, xs, w_gate, w_up, w_down)
    g0, g1 = _sc_combine(ds, slots)

    shg = _shared(x, sw_gate, sw_up, sw_down, shared_gate_w)
    out = _combine(g0, g1, wp[:T], wp[T:], shg)
    return out.reshape(b, s, d)
